# baseline (device time: 157461 ns/iter reference)
import jax
import jax.numpy as jnp
from jax import lax
from jax.experimental import pallas as pl
from jax.experimental.pallas import tpu as pltpu

N_DEV = 4


def kernel(O, Wo):
    B, S, H, D = O.shape
    K = H * D
    N = Wo.shape[1]
    S_out = S // N_DEV

    O3 = O.reshape(B, S, K)

    def body(o_ref, w_ref, out_ref, send_buf, recv_buf, send_sems, recv_sems):
        my_pos = lax.axis_index("i")
        left = (my_pos + N_DEV - 1) % N_DEV
        right = (my_pos + 1) % N_DEV

        barrier_sem = pltpu.get_barrier_semaphore()
        for nbr in (left, right):
            pl.semaphore_signal(
                barrier_sem, inc=1,
                device_id=(nbr,), device_id_type=pl.DeviceIdType.MESH,
            )
        pl.semaphore_wait(barrier_sem, 2)

        w = w_ref[:, :].astype(jnp.bfloat16)

        def partial_chunk(c, out_dtype):
            x = o_ref[:, pl.ds(c * S_out, S_out), :]
            x = x.reshape(B * S_out, K).astype(jnp.bfloat16)
            p = jnp.dot(x, w, preferred_element_type=jnp.float32)
            return p.reshape(B, S_out, N).astype(out_dtype)

        send_buf[:, :, :] = partial_chunk((my_pos + N_DEV - 1) % N_DEV,
                                          jnp.bfloat16)

        for h in range(N_DEV - 1):
            rdma = pltpu.make_async_remote_copy(
                src_ref=send_buf,
                dst_ref=recv_buf.at[h],
                send_sem=send_sems.at[h],
                recv_sem=recv_sems.at[h],
                device_id=(right,),
                device_id_type=pl.DeviceIdType.MESH,
            )
            rdma.start()
            c = (my_pos + N_DEV - 2 - h) % N_DEV
            if h < N_DEV - 2:
                local = partial_chunk(c, jnp.bfloat16)
                rdma.wait()
                send_buf[:, :, :] = recv_buf[h] + local
            else:
                local = partial_chunk(c, jnp.float32)
                rdma.wait()
                out_ref[:, :, :] = recv_buf[h].astype(jnp.float32) + local

    return pl.pallas_call(
        body,
        out_shape=jax.ShapeDtypeStruct((B, S_out, N), jnp.float32),
        in_specs=[
            pl.BlockSpec(memory_space=pltpu.VMEM),
            pl.BlockSpec(memory_space=pltpu.VMEM),
        ],
        out_specs=pl.BlockSpec(memory_space=pltpu.VMEM),
        scratch_shapes=[
            pltpu.VMEM((B, S_out, N), jnp.bfloat16),
            pltpu.VMEM((N_DEV - 1, B, S_out, N), jnp.bfloat16),
            pltpu.SemaphoreType.DMA((N_DEV - 1,)),
            pltpu.SemaphoreType.DMA((N_DEV - 1,)),
        ],
        compiler_params=pltpu.CompilerParams(collective_id=0),
    )(O3, Wo)
